# baseline (speedup 1.0000x reference)
import jax
import jax.numpy as jnp
from jax import lax
from jax.experimental import pallas as pl
from jax.experimental.pallas import tpu as pltpu
from jax.experimental.pallas import tpu_sc as plsc

N_NODES = 100000
N_EDGES = 3200000
F = 16

NC = 2
NS = 16
NW = NC * NS

C = 128
K = 4
U = 6
GROUPS = 198
EPT = GROUPS * C * K
E_PAD = EPT * NW
ROWS_PER_TILE = EPT // C
T_OUTER = GROUPS // U

NPAD = 100096
ZSLICE = NPAD // NS
RB = K * C


def _sc_pass_body(x_hbm, srci_hbm, dsti_hbm, part_out,
                  acc, idx_s, idx_d, rows, isem0, isem1, isem2, gsem, ssem):
  isems = (isem0, isem1, isem2)
  c = lax.axis_index("c")
  s = lax.axis_index("s")
  wid = s * NC + c

  zero16 = jnp.zeros((16,), jnp.float32)

  def zrow(j, carry):
    rows[0, j, :] = zero16
    return carry
  lax.fori_loop(0, RB, zrow, 0)

  zbase = s * ZSLICE
  nfull = ZSLICE // RB
  for t in range(nfull):
    pltpu.sync_copy(rows.at[0], acc.at[pl.ds(zbase + t * RB, RB)])
  rem = ZSLICE - nfull * RB
  if rem:
    pltpu.sync_copy(rows.at[0].at[pl.ds(0, rem)],
                    acc.at[pl.ds(zbase + nfull * RB, rem)])

  plsc.subcore_barrier()

  base_row = wid * ROWS_PER_TILE

  base_e = wid * EPT

  def idx_row(g):
    return jnp.where(g < GROUPS, g, 0)

  def idx_load(sl, g):
    sem = isems[sl]
    pltpu.async_copy(srci_hbm.at[pl.ds(base_e + idx_row(g) * RB, RB)],
                     idx_s.at[sl], sem)
    pltpu.async_copy(dsti_hbm.at[pl.ds(base_row + idx_row(g) * K, K)],
                     idx_d.at[sl], sem)

  def idx_wait(sl):
    sem = isems[sl]
    pltpu.make_async_copy(srci_hbm.at[pl.ds(0, RB)], idx_s.at[sl],
                          sem).wait()
    pltpu.make_async_copy(dsti_hbm.at[pl.ds(0, K)], idx_d.at[sl],
                          sem).wait()

  def fire_gathers(si, sr):
    pltpu.async_copy(x_hbm.at[idx_s.at[si]], rows.at[sr], gsem)

  def wait_gathers(si, sr):
    pltpu.make_async_copy(x_hbm.at[idx_s.at[si]], rows.at[sr], gsem).wait()

  def fire_scatters(si, sr):
    for j in range(K):
      pltpu.async_copy(rows.at[sr].at[pl.ds(j * C, C)],
                       acc.at[idx_d.at[si].at[j]], ssem, add=True)

  def wait_scatters(si, sr):
    for j in range(K):
      pltpu.make_async_copy(rows.at[sr].at[pl.ds(j * C, C)],
                            acc.at[idx_d.at[si].at[j]], ssem).wait()

  pltpu.sync_copy(srci_hbm.at[pl.ds(base_e, RB)], idx_s.at[0])
  pltpu.sync_copy(dsti_hbm.at[pl.ds(base_row, K)], idx_d.at[0])
  idx_load(1, 1)
  fire_gathers(0, 0)

  def outer(t, carry):
    g0 = t * U
    for u in range(U):
      g = g0 + u
      rb, ib = u % 2, u % 3
      ro, i1, i2 = 1 - rb, (u + 1) % 3, (u + 2) % 3
      wait_gathers(ib, rb)
      idx_wait(i1)
      if u == 0:
        @pl.when(t > 0)
        def _():
          wait_scatters(i2, ro)
      else:
        wait_scatters(i2, ro)
      idx_load(i2, g + 2)
      fire_gathers(i1, ro)
      fire_scatters(ib, rb)
    return carry

  lax.fori_loop(0, T_OUTER, outer, 0)

  idx_wait((GROUPS + 1) % 3)
  wait_gathers(GROUPS % 3, GROUPS % 2)
  wait_scatters((GROUPS - 1) % 3, (GROUPS - 1) % 2)

  plsc.subcore_barrier()

  off = c * NPAD + s * ZSLICE
  pltpu.sync_copy(acc.at[pl.ds(s * ZSLICE, ZSLICE)],
                  part_out.at[pl.ds(off, ZSLICE)])


def _make_sc_pass():
  mesh = plsc.VectorSubcoreMesh(core_axis_name="c", subcore_axis_name="s")
  out_type = jax.ShapeDtypeStruct((2 * NPAD, F), jnp.float32)
  scratch = [
      pltpu.VMEM_SHARED((NPAD, F), jnp.float32),
      pltpu.VMEM((3, RB), jnp.int32),
      pltpu.VMEM((3, K, C), jnp.int32),
      pltpu.VMEM((2, RB, F), jnp.float32),
      pltpu.SemaphoreType.DMA,
      pltpu.SemaphoreType.DMA,
      pltpu.SemaphoreType.DMA,
      pltpu.SemaphoreType.DMA,
      pltpu.SemaphoreType.DMA,
  ]
  return pl.kernel(_sc_pass_body, out_type=out_type, mesh=mesh,
                   scratch_types=scratch,
                   compiler_params=pltpu.CompilerParams(
                       use_tc_tiling_on_sc=False),
                   name="sage_agg")


_sc_pass = _make_sc_pass()

KC = 24
CNT_GROUPS = ROWS_PER_TILE // KC
ZB = 512


def _sc_cnt_body(dsti_hbm, cnt_out, cnt_acc, idx_d, ones, zb, ssem):
  c = lax.axis_index("c")
  s = lax.axis_index("s")
  wid = s * NC + c

  zero16 = jnp.zeros((16,), jnp.float32)
  one16 = jnp.ones((16,), jnp.float32)

  def zrow(j, carry):
    zb[j, :] = zero16
    return carry
  lax.fori_loop(0, ZB, zrow, 0)

  def orow(j, carry):
    ones[j, :] = one16
    return carry
  lax.fori_loop(0, C, orow, 0)

  zbase = s * ZSLICE
  nfull = ZSLICE // ZB
  for t in range(nfull):
    pltpu.sync_copy(zb, cnt_acc.at[pl.ds(zbase + t * ZB, ZB)])
  rem = ZSLICE - nfull * ZB
  if rem:
    pltpu.sync_copy(zb.at[pl.ds(0, rem)],
                    cnt_acc.at[pl.ds(zbase + nfull * ZB, rem)])

  plsc.subcore_barrier()

  base_row = wid * ROWS_PER_TILE

  def group(g, carry):
    r = base_row + g * KC
    pltpu.sync_copy(dsti_hbm.at[pl.ds(r, KC)], idx_d)
    sd = []
    for j in range(KC):
      sd.append(pltpu.async_copy(ones, cnt_acc.at[idx_d.at[j]], ssem,
                                 add=True))
    for d in sd:
      d.wait()
    return carry

  lax.fori_loop(0, CNT_GROUPS, group, 0)

  plsc.subcore_barrier()

  off = c * NPAD + s * ZSLICE
  pltpu.sync_copy(cnt_acc.at[pl.ds(s * ZSLICE, ZSLICE)],
                  cnt_out.at[pl.ds(off, ZSLICE)])


def _make_sc_cnt():
  mesh = plsc.VectorSubcoreMesh(core_axis_name="c", subcore_axis_name="s")
  out_type = jax.ShapeDtypeStruct((2 * NPAD, F), jnp.float32)
  scratch = [
      pltpu.VMEM_SHARED((NPAD, F), jnp.float32),
      pltpu.VMEM((KC, C), jnp.int32),
      pltpu.VMEM((C, F), jnp.float32),
      pltpu.VMEM((ZB, F), jnp.float32),
      pltpu.SemaphoreType.DMA,
  ]
  return pl.kernel(_sc_cnt_body, out_type=out_type, mesh=mesh,
                   scratch_types=scratch,
                   compiler_params=pltpu.CompilerParams(
                       use_tc_tiling_on_sc=False),
                   name="sage_cnt")


_sc_cnt = _make_sc_cnt()


PACK = 128 // F
NP8 = NPAD // PACK
NROWS = 12504
N_SC = NROWS * PACK
BT = 4168
GRID = NROWS // BT


def _mean(p0, p1, c0, c1):
  cnt = jnp.maximum(c0[...] + c1[...], 1.0)
  return (p0[...] + p1[...]) / cnt


def _sage(p0, p1, c0, c1, h, wl, bl, wr):
  mean = _mean(p0, p1, c0, c1)
  return jnp.maximum(
      jnp.dot(mean, wl[...], preferred_element_type=jnp.float32) + bl[...]
      + jnp.dot(h[...], wr[...], preferred_element_type=jnp.float32), 0.0)


def _dense_sage_body(p0, p1, c0, c1, h, wl, bl, wr, o):
  o[...] = _sage(p0, p1, c0, c1, h, wl, bl, wr)


def _dense_sage_fc1_body(p0, p1, c0, c1, h, wl, bl, wr, f1, f1b, o):
  t = _sage(p0, p1, c0, c1, h, wl, bl, wr)
  o[...] = jnp.maximum(
      jnp.dot(t, f1[...], preferred_element_type=jnp.float32) + f1b[...], 0.0)


def _dense_sage_fc2_body(p0, p1, c0, c1, h, wl, bl, wr, f2, f2b, gsum, o):
  t = _sage(p0, p1, c0, c1, h, wl, bl, wr)
  u = jnp.maximum(
      jnp.dot(t, f2[...], preferred_element_type=jnp.float32) + f2b[...], 0.0)
  lane = jax.lax.broadcasted_iota(jnp.int32, u.shape, 1)
  m = ((lane % F) < 8).astype(jnp.float32)
  e = jnp.exp(u - jnp.max(u, axis=1, keepdims=True)) * m
  s = jnp.dot(e, gsum[...], preferred_element_type=jnp.float32)
  o[...] = e / s


def _row_spec():
  return pl.BlockSpec((BT, 128), lambda i: (i, 0))


def _full_spec(shape):
  return pl.BlockSpec(shape, lambda i: (0,) * len(shape))


def _dense_call(body, p0, p1, c0, c1, h, *weights):
  in_specs = [_row_spec()] * 5 + [_full_spec(w.shape) for w in weights]
  return pl.pallas_call(
      body,
      grid=(GRID,),
      in_specs=in_specs,
      out_specs=_row_spec(),
      out_shape=jax.ShapeDtypeStruct((NROWS, 128), jnp.float32),
  )(p0, p1, c0, c1, h, *weights)


def kernel(x, edge_index, Wl10, Wr10, Wl11, Wr11, Wl20, Wr20, Wl21, Wr21,
           bl10, bl11, bl20, bl21, fc1W, fc1b, fc2W, fc2b):
  f32 = jnp.float32
  src = edge_index[0]
  dst = edge_index[1]
  pad = E_PAD - N_EDGES
  srci = jnp.concatenate([src, jnp.zeros((pad,), jnp.int32)])
  dsti = jnp.concatenate([dst, jnp.full((pad,), N_NODES, jnp.int32)])
  dsti = dsti.reshape(E_PAD // C, C)

  eye8 = jnp.eye(PACK, dtype=f32)

  def bd(w):
    return jnp.kron(eye8, w.astype(f32))

  def tile_b(b):
    return jnp.tile(b.astype(f32), PACK).reshape(1, 128)

  def packed_halves(slab):
    pp = slab.reshape(2 * NP8, 128)
    return pp[:NROWS], pp[NP8:NP8 + NROWS]

  def to_sc(hp):
    return hp.reshape(N_SC, F)

  xp = jnp.concatenate(
      [x.reshape(N_NODES // PACK, 128), jnp.zeros((4, 128), f32)])
  x_sc = to_sc(xp)

  cnt2 = _sc_cnt(dsti)
  c0, c1 = packed_halves(cnt2)

  part1 = _sc_pass(x_sc, srci, dsti)
  h1 = _dense_call(_dense_sage_body, *packed_halves(part1), c0, c1, xp,
                   bd(Wl10.T), tile_b(bl10), bd(Wr10.T))

  part2 = _sc_pass(to_sc(h1), srci, dsti)
  h3 = _dense_call(_dense_sage_fc1_body, *packed_halves(part2), c0, c1, h1,
                   bd(Wl11.T), tile_b(bl11), bd(Wr11.T),
                   bd(fc1W.T), tile_b(fc1b))

  part3 = _sc_pass(to_sc(h3), srci, dsti)
  h4 = _dense_call(_dense_sage_body, *packed_halves(part3), c0, c1, h3,
                   bd(Wl20.T), tile_b(bl20), bd(Wr20.T))

  part4 = _sc_pass(to_sc(h4), srci, dsti)
  p2 = jnp.zeros((F, F), f32).at[:8, :8].set(fc2W.T.astype(f32))
  f2b = jnp.concatenate([fc2b.astype(f32), jnp.zeros((8,), f32)])
  gsum = bd(jnp.ones((F, F), f32))
  outp = _dense_call(_dense_sage_fc2_body, *packed_halves(part4), c0, c1, h4,
                     bd(Wl21.T), tile_b(bl21), bd(Wr21.T),
                     bd(p2), tile_b(f2b), gsum)
  return outp[:N_NODES // PACK].reshape(N_NODES, F)[:, :8]


# speedup vs baseline: 1.0129x; 1.0129x over previous
import jax
import jax.numpy as jnp
from jax import lax
from jax.experimental import pallas as pl
from jax.experimental.pallas import tpu as pltpu
from jax.experimental.pallas import tpu_sc as plsc

N_NODES = 100000
N_EDGES = 3200000
F = 16

NC = 2
NS = 16
NW = NC * NS

C = 128
K = 4
U = 6
GROUPS = 198
EPT = GROUPS * C * K
E_PAD = EPT * NW
ROWS_PER_TILE = EPT // C
T_OUTER = GROUPS // U

NPAD = 100096
ZSLICE = NPAD // NS
RB = K * C


def _sc_pass_body(with_cnt, x_hbm, srci_hbm, dsti_hbm, part_out, *rest):
  if with_cnt:
    (cnt_out, acc, idx_s, idx_d, rows, isem0, isem1, isem2, gsem, ssem,
     cnt_acc, ones, zb, csem) = rest
  else:
    acc, idx_s, idx_d, rows, isem0, isem1, isem2, gsem, ssem = rest
  isems = (isem0, isem1, isem2)
  c = lax.axis_index("c")
  s = lax.axis_index("s")
  wid = s * NC + c

  zero16 = jnp.zeros((16,), jnp.float32)

  def zrow(j, carry):
    rows[0, j, :] = zero16
    return carry
  lax.fori_loop(0, RB, zrow, 0)

  zbase = s * ZSLICE
  nfull = ZSLICE // RB
  for t in range(nfull):
    pltpu.sync_copy(rows.at[0], acc.at[pl.ds(zbase + t * RB, RB)])
  rem = ZSLICE - nfull * RB
  if rem:
    pltpu.sync_copy(rows.at[0].at[pl.ds(0, rem)],
                    acc.at[pl.ds(zbase + nfull * RB, rem)])

  if with_cnt:
    def zc(j, carry):
      zb[pl.ds(j * 16, 16)] = zero16
      return carry
    lax.fori_loop(0, 1024 // 16, zc, 0)
    for j in range(C // 16):
      ones[pl.ds(j * 16, 16)] = jnp.ones((16,), jnp.float32)
    for t in range(ZSLICE // 1024):
      pltpu.sync_copy(zb, cnt_acc.at[pl.ds(zbase + t * 1024, 1024)])
    crem = ZSLICE % 1024
    pltpu.sync_copy(zb.at[pl.ds(0, crem)],
                    cnt_acc.at[pl.ds(zbase + ZSLICE - crem, crem)])

  plsc.subcore_barrier()

  base_row = wid * ROWS_PER_TILE

  base_e = wid * EPT

  def idx_row(g):
    return jnp.where(g < GROUPS, g, 0)

  def idx_load(sl, g):
    sem = isems[sl]
    pltpu.async_copy(srci_hbm.at[pl.ds(base_e + idx_row(g) * RB, RB)],
                     idx_s.at[sl], sem)
    pltpu.async_copy(dsti_hbm.at[pl.ds(base_row + idx_row(g) * K, K)],
                     idx_d.at[sl], sem)

  def idx_wait(sl):
    sem = isems[sl]
    pltpu.make_async_copy(srci_hbm.at[pl.ds(0, RB)], idx_s.at[sl],
                          sem).wait()
    pltpu.make_async_copy(dsti_hbm.at[pl.ds(0, K)], idx_d.at[sl],
                          sem).wait()

  def fire_gathers(si, sr):
    pltpu.async_copy(x_hbm.at[idx_s.at[si]], rows.at[sr], gsem)

  def wait_gathers(si, sr):
    pltpu.make_async_copy(x_hbm.at[idx_s.at[si]], rows.at[sr], gsem).wait()

  def fire_scatters(si, sr):
    for j in range(K):
      pltpu.async_copy(rows.at[sr].at[pl.ds(j * C, C)],
                       acc.at[idx_d.at[si].at[j]], ssem, add=True)
      if with_cnt:
        pltpu.async_copy(ones, cnt_acc.at[idx_d.at[si].at[j]], csem,
                         add=True)

  def wait_scatters(si, sr):
    for j in range(K):
      pltpu.make_async_copy(rows.at[sr].at[pl.ds(j * C, C)],
                            acc.at[idx_d.at[si].at[j]], ssem).wait()
      if with_cnt:
        pltpu.make_async_copy(ones, cnt_acc.at[idx_d.at[si].at[j]],
                              csem).wait()

  pltpu.sync_copy(srci_hbm.at[pl.ds(base_e, RB)], idx_s.at[0])
  pltpu.sync_copy(dsti_hbm.at[pl.ds(base_row, K)], idx_d.at[0])
  idx_load(1, 1)
  fire_gathers(0, 0)

  def outer(t, carry):
    g0 = t * U
    for u in range(U):
      g = g0 + u
      rb, ib = u % 2, u % 3
      ro, i1, i2 = 1 - rb, (u + 1) % 3, (u + 2) % 3
      wait_gathers(ib, rb)
      idx_wait(i1)
      if u == 0:
        @pl.when(t > 0)
        def _():
          wait_scatters(i2, ro)
      else:
        wait_scatters(i2, ro)
      idx_load(i2, g + 2)
      fire_gathers(i1, ro)
      fire_scatters(ib, rb)
    return carry

  lax.fori_loop(0, T_OUTER, outer, 0)

  idx_wait((GROUPS + 1) % 3)
  wait_gathers(GROUPS % 3, GROUPS % 2)
  wait_scatters((GROUPS - 1) % 3, (GROUPS - 1) % 2)

  plsc.subcore_barrier()

  off = c * NPAD + s * ZSLICE
  pltpu.sync_copy(acc.at[pl.ds(s * ZSLICE, ZSLICE)],
                  part_out.at[pl.ds(off, ZSLICE)])
  if with_cnt:
    @pl.when(s == 0)
    def _():
      pltpu.sync_copy(cnt_acc, cnt_out.at[pl.ds(c * NPAD, NPAD)])


def _make_sc_pass(with_cnt):
  mesh = plsc.VectorSubcoreMesh(core_axis_name="c", subcore_axis_name="s")
  out_type = [jax.ShapeDtypeStruct((2 * NPAD, F), jnp.float32)]
  scratch = [
      pltpu.VMEM_SHARED((NPAD, F), jnp.float32),
      pltpu.VMEM((3, RB), jnp.int32),
      pltpu.VMEM((3, K, C), jnp.int32),
      pltpu.VMEM((2, RB, F), jnp.float32),
      pltpu.SemaphoreType.DMA,
      pltpu.SemaphoreType.DMA,
      pltpu.SemaphoreType.DMA,
      pltpu.SemaphoreType.DMA,
      pltpu.SemaphoreType.DMA,
  ]
  if with_cnt:
    out_type.append(jax.ShapeDtypeStruct((2 * NPAD,), jnp.float32))
    scratch += [
        pltpu.VMEM_SHARED((NPAD,), jnp.float32),
        pltpu.VMEM((C,), jnp.float32),
        pltpu.VMEM((1024,), jnp.float32),
        pltpu.SemaphoreType.DMA,
    ]

  def body(*args):
    _sc_pass_body(with_cnt, *args)

  return pl.kernel(body, out_type=tuple(out_type) if with_cnt else out_type[0],
                   mesh=mesh, scratch_types=scratch,
                   compiler_params=pltpu.CompilerParams(
                       use_tc_tiling_on_sc=False),
                   name="sage_agg_cnt" if with_cnt else "sage_agg")


_sc_pass = _make_sc_pass(False)
_sc_pass_cnt = _make_sc_pass(True)


PACK = 128 // F
NP8 = NPAD // PACK
NROWS = 12504
N_SC = NROWS * PACK
BT = 4168
GRID = NROWS // BT


def _mean(p0, p1, c0, c1, rsel):
  cnt = jnp.dot(c0[...] + c1[...], rsel[...],
                preferred_element_type=jnp.float32)
  return (p0[...] + p1[...]) / jnp.maximum(cnt, 1.0)


def _sage(p0, p1, c0, c1, h, rsel, wl, bl, wr):
  mean = _mean(p0, p1, c0, c1, rsel)
  return jnp.maximum(
      jnp.dot(mean, wl[...], preferred_element_type=jnp.float32) + bl[...]
      + jnp.dot(h[...], wr[...], preferred_element_type=jnp.float32), 0.0)


def _dense_sage_body(p0, p1, c0, c1, h, rsel, wl, bl, wr, o):
  o[...] = _sage(p0, p1, c0, c1, h, rsel, wl, bl, wr)


def _dense_sage_fc1_body(p0, p1, c0, c1, h, rsel, wl, bl, wr, f1, f1b, o):
  t = _sage(p0, p1, c0, c1, h, rsel, wl, bl, wr)
  o[...] = jnp.maximum(
      jnp.dot(t, f1[...], preferred_element_type=jnp.float32) + f1b[...], 0.0)


def _dense_sage_fc2_body(p0, p1, c0, c1, h, rsel, wl, bl, wr, f2, f2b, gsum,
                         o):
  t = _sage(p0, p1, c0, c1, h, rsel, wl, bl, wr)
  u = jnp.maximum(
      jnp.dot(t, f2[...], preferred_element_type=jnp.float32) + f2b[...], 0.0)
  lane = jax.lax.broadcasted_iota(jnp.int32, u.shape, 1)
  m = ((lane % F) < 8).astype(jnp.float32)
  e = jnp.exp(u - jnp.max(u, axis=1, keepdims=True)) * m
  s = jnp.dot(e, gsum[...], preferred_element_type=jnp.float32)
  o[...] = e / s


def _row_spec(width=128):
  return pl.BlockSpec((BT, width), lambda i: (i, 0))


def _full_spec(shape):
  return pl.BlockSpec(shape, lambda i: (0,) * len(shape))


def _dense_call(body, p0, p1, c0, c1, h, *weights):
  in_specs = [_row_spec(), _row_spec(), _row_spec(PACK), _row_spec(PACK),
              _row_spec()] + [_full_spec(w.shape) for w in weights]
  return pl.pallas_call(
      body,
      grid=(GRID,),
      in_specs=in_specs,
      out_specs=_row_spec(),
      out_shape=jax.ShapeDtypeStruct((NROWS, 128), jnp.float32),
  )(p0, p1, c0, c1, h, *weights)


def kernel(x, edge_index, Wl10, Wr10, Wl11, Wr11, Wl20, Wr20, Wl21, Wr21,
           bl10, bl11, bl20, bl21, fc1W, fc1b, fc2W, fc2b):
  f32 = jnp.float32
  src = edge_index[0]
  dst = edge_index[1]
  pad = E_PAD - N_EDGES
  srci = jnp.concatenate([src, jnp.zeros((pad,), jnp.int32)])
  dsti = jnp.concatenate([dst, jnp.full((pad,), N_NODES, jnp.int32)])
  dsti = dsti.reshape(E_PAD // C, C)

  eye8 = jnp.eye(PACK, dtype=f32)

  def bd(w):
    return jnp.kron(eye8, w.astype(f32))

  def tile_b(b):
    return jnp.tile(b.astype(f32), PACK).reshape(1, 128)

  def packed_halves(slab):
    pp = slab.reshape(2 * NP8, 128)
    return pp[:NROWS], pp[NP8:NP8 + NROWS]

  def to_sc(hp):
    return hp.reshape(N_SC, F)

  xp = jnp.concatenate(
      [x.reshape(N_NODES // PACK, 128), jnp.zeros((4, 128), f32)])
  x_sc = to_sc(xp)

  rsel = jnp.kron(eye8, jnp.ones((1, F), f32))

  part1, cnt2 = _sc_pass_cnt(x_sc, srci, dsti)
  c0 = cnt2[:N_SC].reshape(NROWS, PACK)
  c1 = cnt2[NPAD:NPAD + N_SC].reshape(NROWS, PACK)

  h1 = _dense_call(_dense_sage_body, *packed_halves(part1), c0, c1, xp,
                   rsel, bd(Wl10.T), tile_b(bl10), bd(Wr10.T))

  part2 = _sc_pass(to_sc(h1), srci, dsti)
  h3 = _dense_call(_dense_sage_fc1_body, *packed_halves(part2), c0, c1, h1,
                   rsel, bd(Wl11.T), tile_b(bl11), bd(Wr11.T),
                   bd(fc1W.T), tile_b(fc1b))

  part3 = _sc_pass(to_sc(h3), srci, dsti)
  h4 = _dense_call(_dense_sage_body, *packed_halves(part3), c0, c1, h3,
                   rsel, bd(Wl20.T), tile_b(bl20), bd(Wr20.T))

  part4 = _sc_pass(to_sc(h4), srci, dsti)
  p2 = jnp.zeros((F, F), f32).at[:8, :8].set(fc2W.T.astype(f32))
  f2b = jnp.concatenate([fc2b.astype(f32), jnp.zeros((8,), f32)])
  gsum = bd(jnp.ones((F, F), f32))
  outp = _dense_call(_dense_sage_fc2_body, *packed_halves(part4), c0, c1, h4,
                     rsel, bd(Wl21.T), tile_b(bl21), bd(Wr21.T),
                     bd(p2), tile_b(f2b), gsum)
  return outp[:N_NODES // PACK].reshape(N_NODES, F)[:, :8]
